# Initial kernel scaffold; baseline (speedup 1.0000x reference)
import jax
import jax.numpy as jnp
from jax.experimental import pallas as pl


def kernel(x, embeddings):
    raise NotImplementedError("write your pallas kernel here")


# speedup vs baseline: 1.7330x; 1.7330x over previous
import functools

import jax
import jax.numpy as jnp
from jax import lax
from jax.experimental import pallas as pl
from jax.experimental.pallas import tpu as pltpu
from jax.experimental.pallas import tpu_sc as plsc

TOKENS = 8192
DIM = 256
CODES = 8192
TBLK = 256
NBLK = TOKENS // TBLK
LANES = 128
NCHUNK = CODES // LANES

SC_CORES = 2
SC_SUBCORES = 16
SC_WORKERS = SC_CORES * SC_SUBCORES
ROWS_PER_WORKER = TOKENS // SC_WORKERS
IDX_CHUNKS = ROWS_PER_WORKER // LANES


def _argmin_body(flat_ref, emb_ref, embcol_ref, idx_ref, embt_ref,
                 esq_ref, ebf_ref):
    i = pl.program_id(0)

    @pl.when(i == 0)
    def _init():
        e = emb_ref[...]
        esq_ref[...] = jnp.sum(e * e, axis=0, keepdims=True)
        ebf_ref[...] = e.astype(jnp.bfloat16)

    eblk = embcol_ref[...]
    embt_ref[...] = (
        jnp.transpose(eblk).astype(jnp.bfloat16).astype(jnp.float32))

    f = flat_ref[...]
    sq = f * f
    fsq = jnp.sum(sq[:, :LANES] + sq[:, LANES:], axis=1, keepdims=True)

    fbf = f.astype(jnp.bfloat16)
    sim = jnp.dot(fbf, ebf_ref[...],
                  preferred_element_type=jnp.float32)
    dist = (fsq + esq_ref[...]) - 2.0 * sim

    lane = lax.broadcasted_iota(jnp.int32, (TBLK, LANES), 1)
    runval = dist[:, :LANES]
    runidx = lane
    for c in range(1, NCHUNK):
        d = dist[:, c * LANES:(c + 1) * LANES]
        m = d < runval
        runval = jnp.where(m, d, runval)
        runidx = jnp.where(m, lane + c * LANES, runidx)

    fmin = jnp.min(runval, axis=1, keepdims=True)
    cand = jnp.where(runval == fmin, runidx, jnp.int32(0x7FFFFFFF))
    fidx = jnp.min(cand, axis=1, keepdims=True)
    idx_ref[...] = fidx[None]


@jax.jit
def _argmin_call(flat, embeddings):
    return pl.pallas_call(
        _argmin_body,
        grid=(NBLK,),
        in_specs=[
            pl.BlockSpec((TBLK, DIM), lambda i: (i, 0)),
            pl.BlockSpec((DIM, CODES), lambda i: (0, 0)),
            pl.BlockSpec((DIM, TBLK), lambda i: (0, i)),
        ],
        out_specs=[
            pl.BlockSpec((1, TBLK, 1), lambda i: (i, 0, 0)),
            pl.BlockSpec((TBLK, DIM), lambda i: (i, 0)),
        ],
        out_shape=[
            jax.ShapeDtypeStruct((NBLK, TBLK, 1), jnp.int32),
            jax.ShapeDtypeStruct((TOKENS, DIM), jnp.float32),
        ],
        scratch_shapes=[
            pltpu.VMEM((1, CODES), jnp.float32),
            pltpu.VMEM((DIM, CODES), jnp.bfloat16),
        ],
    )(flat, embeddings, embeddings)


def _gather_call(embt, idx):
    mesh = plsc.VectorSubcoreMesh(core_axis_name="c", subcore_axis_name="s")

    @functools.partial(
        pl.kernel,
        mesh=mesh,
        out_type=jax.ShapeDtypeStruct((TOKENS, DIM), jnp.float32),
        scratch_types=[
            pltpu.VMEM((IDX_CHUNKS, LANES), jnp.int32),
            pltpu.VMEM((ROWS_PER_WORKER, DIM), jnp.float32),
            pltpu.SemaphoreType.DMA,
        ],
    )
    def k(table_hbm, idx_hbm, out_hbm, idx_v, rows_v, sem):
        wid = lax.axis_index("s") * SC_CORES + lax.axis_index("c")
        pltpu.sync_copy(idx_hbm.at[wid], idx_v)
        copies = [
            pltpu.async_copy(table_hbm.at[idx_v.at[j]],
                             rows_v.at[pl.ds(j * LANES, LANES)], sem)
            for j in range(IDX_CHUNKS)
        ]
        for cp in copies:
            cp.wait()
        pltpu.sync_copy(rows_v, out_hbm.at[pl.ds(wid * ROWS_PER_WORKER,
                                                 ROWS_PER_WORKER)])

    return k(embt, idx)


def kernel(x, embeddings):
    flat = x.reshape(TOKENS, DIM)
    idx, embt = _argmin_call(flat, embeddings)
    idx = idx.reshape(SC_WORKERS, IDX_CHUNKS, LANES)
    quantized = _gather_call(embt, idx)
    return quantized.reshape(x.shape)


# speedup vs baseline: 1.9459x; 1.1228x over previous
import functools

import jax
import jax.numpy as jnp
from jax import lax
from jax.experimental import pallas as pl
from jax.experimental.pallas import tpu as pltpu
from jax.experimental.pallas import tpu_sc as plsc

TOKENS = 8192
DIM = 256
CODES = 8192
TBLK = 1024
NBLK = TOKENS // TBLK
LANES = 128
CBLK = 1024
NCHUNK = CODES // CBLK

SC_CORES = 2
SC_SUBCORES = 16
SC_WORKERS = SC_CORES * SC_SUBCORES
ROWS_PER_WORKER = TOKENS // SC_WORKERS
IDX_CHUNKS = ROWS_PER_WORKER // LANES


def _argmin_body(flat_ref, emb_ref, embcol_ref, idx_ref, embt_ref,
                 esq_ref, ebf_ref):
    i = pl.program_id(0)

    @pl.when(i == 0)
    def _init():
        e = emb_ref[...]
        esq_ref[...] = jnp.sum(e * e, axis=0, keepdims=True)
        ebf_ref[...] = e.astype(jnp.bfloat16)

    eblk = embcol_ref[...]
    embt_ref[...] = (
        jnp.transpose(eblk).astype(jnp.bfloat16).astype(jnp.float32))

    f = flat_ref[...]
    sq = f * f
    fsq = jnp.sum(sq[:, :LANES] + sq[:, LANES:], axis=1, keepdims=True)

    fbf2 = f.astype(jnp.bfloat16) * jnp.bfloat16(2.0)
    esq = esq_ref[...]

    runval = None
    runchunk = None
    for c in range(NCHUNK):
        sl = slice(c * CBLK, (c + 1) * CBLK)
        sim2 = jnp.dot(fbf2, ebf_ref[:, sl],
                       preferred_element_type=jnp.float32)
        d = (fsq + esq[:, sl]) - sim2
        if c == 0:
            runval = d
            runchunk = jnp.zeros((TBLK, CBLK), jnp.int32)
        else:
            m = d < runval
            runval = jnp.minimum(runval, d)
            runchunk = jnp.where(m, jnp.int32(c), runchunk)

    lane = lax.broadcasted_iota(jnp.int32, (TBLK, CBLK), 1)
    runidx = runchunk * CBLK + lane
    fmin = jnp.min(runval, axis=1, keepdims=True)
    cand = jnp.where(runval == fmin, runidx, jnp.int32(0x7FFFFFFF))
    fidx = jnp.min(cand, axis=1, keepdims=True)
    idx_ref[...] = fidx[None]


@jax.jit
def _argmin_call(flat, embeddings):
    return pl.pallas_call(
        _argmin_body,
        grid=(NBLK,),
        in_specs=[
            pl.BlockSpec((TBLK, DIM), lambda i: (i, 0)),
            pl.BlockSpec((DIM, CODES), lambda i: (0, 0)),
            pl.BlockSpec((DIM, TBLK), lambda i: (0, i)),
        ],
        out_specs=[
            pl.BlockSpec((1, TBLK, 1), lambda i: (i, 0, 0)),
            pl.BlockSpec((TBLK, DIM), lambda i: (i, 0)),
        ],
        out_shape=[
            jax.ShapeDtypeStruct((NBLK, TBLK, 1), jnp.int32),
            jax.ShapeDtypeStruct((TOKENS, DIM), jnp.float32),
        ],
        scratch_shapes=[
            pltpu.VMEM((1, CODES), jnp.float32),
            pltpu.VMEM((DIM, CODES), jnp.bfloat16),
        ],
    )(flat, embeddings, embeddings)


def _gather_call(embt, idx):
    mesh = plsc.VectorSubcoreMesh(core_axis_name="c", subcore_axis_name="s")

    @functools.partial(
        pl.kernel,
        mesh=mesh,
        out_type=jax.ShapeDtypeStruct((TOKENS, DIM), jnp.float32),
        scratch_types=[
            pltpu.VMEM((IDX_CHUNKS, LANES), jnp.int32),
            pltpu.VMEM((ROWS_PER_WORKER, DIM), jnp.float32),
            pltpu.SemaphoreType.DMA,
        ],
    )
    def k(table_hbm, idx_hbm, out_hbm, idx_v, rows_v, sem):
        wid = lax.axis_index("s") * SC_CORES + lax.axis_index("c")
        pltpu.sync_copy(idx_hbm.at[wid], idx_v)
        copies = [
            pltpu.async_copy(table_hbm.at[idx_v.at[j]],
                             rows_v.at[pl.ds(j * LANES, LANES)], sem)
            for j in range(IDX_CHUNKS)
        ]
        for cp in copies:
            cp.wait()
        pltpu.sync_copy(rows_v, out_hbm.at[pl.ds(wid * ROWS_PER_WORKER,
                                                 ROWS_PER_WORKER)])

    return k(embt, idx)


def kernel(x, embeddings):
    flat = x.reshape(TOKENS, DIM)
    idx, embt = _argmin_call(flat, embeddings)
    idx = idx.reshape(SC_WORKERS, IDX_CHUNKS, LANES)
    quantized = _gather_call(embt, idx)
    return quantized.reshape(x.shape)


# speedup vs baseline: 2.0093x; 1.0326x over previous
import functools

import jax
import jax.numpy as jnp
from jax import lax
from jax.experimental import pallas as pl
from jax.experimental.pallas import tpu as pltpu
from jax.experimental.pallas import tpu_sc as plsc

TOKENS = 8192
DIM = 256
CODES = 8192
TBLK = 1024
NBLK = TOKENS // TBLK
LANES = 128
CBLK = 1024
NCHUNK = CODES // CBLK

SC_CORES = 2
SC_SUBCORES = 16
SC_WORKERS = SC_CORES * SC_SUBCORES
ROWS_PER_WORKER = TOKENS // SC_WORKERS
IDX_CHUNKS = ROWS_PER_WORKER // LANES


def _argmin_body(flat_ref, emb_ref, embcol_ref, idx_ref, embt_ref,
                 esq_ref, ebf_ref):
    i = pl.program_id(0)

    @pl.when(i == 0)
    def _init():
        e = emb_ref[...]
        esq_ref[...] = jnp.sum(e * e, axis=0, keepdims=True)
        ebf_ref[...] = e.astype(jnp.bfloat16)

    eblk = embcol_ref[...]
    embt_ref[...] = (
        jnp.transpose(eblk).astype(jnp.bfloat16).astype(jnp.float32))

    f = flat_ref[...]
    sq = f * f
    fsq = jnp.sum(sq[:, :LANES] + sq[:, LANES:], axis=1, keepdims=True)

    fbf2 = f.astype(jnp.bfloat16) * jnp.bfloat16(2.0)
    esq = esq_ref[...]

    runval = None
    runchunk = None
    for c in range(NCHUNK):
        sl = slice(c * CBLK, (c + 1) * CBLK)
        sim2 = jnp.dot(fbf2, ebf_ref[:, sl],
                       preferred_element_type=jnp.float32)
        d = (fsq + esq[:, sl]) - sim2
        if c == 0:
            runval = d
            runchunk = jnp.zeros((TBLK, CBLK), jnp.int32)
        else:
            m = d < runval
            runval = jnp.minimum(runval, d)
            runchunk = jnp.where(m, jnp.int32(c), runchunk)

    lane = lax.broadcasted_iota(jnp.int32, (TBLK, CBLK), 1)
    runidx = runchunk * CBLK + lane
    fmin = jnp.min(runval, axis=1, keepdims=True)
    cand = jnp.where(runval == fmin, runidx, jnp.int32(0x7FFFFFFF))
    fidx = jnp.min(cand, axis=1, keepdims=True)
    idx_ref[...] = fidx.reshape(TBLK // ROWS_PER_WORKER, IDX_CHUNKS, LANES)


@jax.jit
def _argmin_call(flat, embeddings):
    return pl.pallas_call(
        _argmin_body,
        grid=(NBLK,),
        in_specs=[
            pl.BlockSpec((TBLK, DIM), lambda i: (i, 0)),
            pl.BlockSpec((DIM, CODES), lambda i: (0, 0)),
            pl.BlockSpec((DIM, TBLK), lambda i: (0, i)),
        ],
        out_specs=[
            pl.BlockSpec((TBLK // ROWS_PER_WORKER, IDX_CHUNKS, LANES),
                         lambda i: (i, 0, 0)),
            pl.BlockSpec((TBLK, DIM), lambda i: (i, 0)),
        ],
        out_shape=[
            jax.ShapeDtypeStruct((SC_WORKERS, IDX_CHUNKS, LANES), jnp.int32),
            jax.ShapeDtypeStruct((TOKENS, DIM), jnp.float32),
        ],
        scratch_shapes=[
            pltpu.VMEM((1, CODES), jnp.float32),
            pltpu.VMEM((DIM, CODES), jnp.bfloat16),
        ],
    )(flat, embeddings, embeddings)


def _gather_call(embt, idx):
    mesh = plsc.VectorSubcoreMesh(core_axis_name="c", subcore_axis_name="s")

    @functools.partial(
        pl.kernel,
        mesh=mesh,
        out_type=jax.ShapeDtypeStruct((TOKENS, DIM), jnp.float32),
        scratch_types=[
            pltpu.VMEM((IDX_CHUNKS, LANES), jnp.int32),
            pltpu.VMEM((ROWS_PER_WORKER, DIM), jnp.float32),
            pltpu.SemaphoreType.DMA,
        ],
    )
    def k(table_hbm, idx_hbm, out_hbm, idx_v, rows_v, sem):
        wid = lax.axis_index("s") * SC_CORES + lax.axis_index("c")
        pltpu.sync_copy(idx_hbm.at[wid], idx_v)
        copies = [
            pltpu.async_copy(table_hbm.at[idx_v.at[j]],
                             rows_v.at[pl.ds(j * LANES, LANES)], sem)
            for j in range(IDX_CHUNKS)
        ]
        for cp in copies:
            cp.wait()
        pltpu.sync_copy(rows_v, out_hbm.at[pl.ds(wid * ROWS_PER_WORKER,
                                                 ROWS_PER_WORKER)])

    return k(embt, idx)


def kernel(x, embeddings):
    flat = x.reshape(TOKENS, DIM)
    idx, embt = _argmin_call(flat, embeddings)
    quantized = _gather_call(embt, idx)
    return quantized.reshape(x.shape)


# speedup vs baseline: 2.2422x; 1.1159x over previous
import functools

import jax
import jax.numpy as jnp
from jax import lax
from jax.experimental import pallas as pl
from jax.experimental.pallas import tpu as pltpu
from jax.experimental.pallas import tpu_sc as plsc

TOKENS = 8192
DIM = 256
CODES = 8192
TBLK = 1024
NBLK = TOKENS // TBLK
LANES = 128
CBLK = 256
NCHUNK = CODES // CBLK

SC_CORES = 2
SC_SUBCORES = 16
SC_WORKERS = SC_CORES * SC_SUBCORES
ROWS_PER_WORKER = TOKENS // SC_WORKERS
IDX_CHUNKS = ROWS_PER_WORKER // LANES


def _argmin_body(flat_ref, emb_ref, embcol_ref, idx_ref, embt_ref,
                 esq_ref, ebf_ref):
    i = pl.program_id(0)

    @pl.when(i == 0)
    def _init():
        e = emb_ref[...]
        esq_ref[...] = jnp.sum(e * e, axis=0, keepdims=True)
        ebf_ref[...] = e.astype(jnp.bfloat16)

    eblk = embcol_ref[...]
    embt_ref[...] = (
        jnp.transpose(eblk).astype(jnp.bfloat16).astype(jnp.float32))

    f = flat_ref[...]
    sq = f * f
    fsq = jnp.sum(sq[:, :LANES] + sq[:, LANES:], axis=1, keepdims=True)

    fbf2 = f.astype(jnp.bfloat16) * jnp.bfloat16(2.0)
    esq = esq_ref[...]

    runval = None
    runchunk = None
    for c in range(NCHUNK):
        sl = slice(c * CBLK, (c + 1) * CBLK)
        sim2 = jnp.dot(fbf2, ebf_ref[:, sl],
                       preferred_element_type=jnp.float32)
        d = (fsq + esq[:, sl]) - sim2
        if c == 0:
            runval = d
            runchunk = jnp.zeros((TBLK, CBLK), jnp.int32)
        else:
            m = d < runval
            runval = jnp.minimum(runval, d)
            runchunk = jnp.where(m, jnp.int32(c), runchunk)

    lane = lax.broadcasted_iota(jnp.int32, (TBLK, CBLK), 1)
    runidx = runchunk * CBLK + lane
    fmin = jnp.min(runval, axis=1, keepdims=True)
    cand = jnp.where(runval == fmin, runidx, jnp.int32(0x7FFFFFFF))
    fidx = jnp.min(cand, axis=1, keepdims=True)
    idx_ref[...] = fidx.reshape(TBLK // ROWS_PER_WORKER, IDX_CHUNKS, LANES)


@jax.jit
def _argmin_call(flat, embeddings):
    return pl.pallas_call(
        _argmin_body,
        grid=(NBLK,),
        in_specs=[
            pl.BlockSpec((TBLK, DIM), lambda i: (i, 0)),
            pl.BlockSpec((DIM, CODES), lambda i: (0, 0)),
            pl.BlockSpec((DIM, TBLK), lambda i: (0, i)),
        ],
        out_specs=[
            pl.BlockSpec((TBLK // ROWS_PER_WORKER, IDX_CHUNKS, LANES),
                         lambda i: (i, 0, 0)),
            pl.BlockSpec((TBLK, DIM), lambda i: (i, 0)),
        ],
        out_shape=[
            jax.ShapeDtypeStruct((SC_WORKERS, IDX_CHUNKS, LANES), jnp.int32),
            jax.ShapeDtypeStruct((TOKENS, DIM), jnp.float32),
        ],
        scratch_shapes=[
            pltpu.VMEM((1, CODES), jnp.float32),
            pltpu.VMEM((DIM, CODES), jnp.bfloat16),
        ],
    )(flat, embeddings, embeddings)


def _gather_call(embt, idx):
    mesh = plsc.VectorSubcoreMesh(core_axis_name="c", subcore_axis_name="s")

    @functools.partial(
        pl.kernel,
        mesh=mesh,
        out_type=jax.ShapeDtypeStruct((TOKENS, DIM), jnp.float32),
        scratch_types=[
            pltpu.VMEM((IDX_CHUNKS, LANES), jnp.int32),
            pltpu.VMEM((ROWS_PER_WORKER, DIM), jnp.float32),
            pltpu.SemaphoreType.DMA,
        ],
    )
    def k(table_hbm, idx_hbm, out_hbm, idx_v, rows_v, sem):
        wid = lax.axis_index("s") * SC_CORES + lax.axis_index("c")
        pltpu.sync_copy(idx_hbm.at[wid], idx_v)
        copies = [
            pltpu.async_copy(table_hbm.at[idx_v.at[j]],
                             rows_v.at[pl.ds(j * LANES, LANES)], sem)
            for j in range(IDX_CHUNKS)
        ]
        for cp in copies:
            cp.wait()
        pltpu.sync_copy(rows_v, out_hbm.at[pl.ds(wid * ROWS_PER_WORKER,
                                                 ROWS_PER_WORKER)])

    return k(embt, idx)


def kernel(x, embeddings):
    flat = x.reshape(TOKENS, DIM)
    idx, embt = _argmin_call(flat, embeddings)
    quantized = _gather_call(embt, idx)
    return quantized.reshape(x.shape)


# speedup vs baseline: 2.3357x; 1.0417x over previous
import functools

import jax
import jax.numpy as jnp
from jax import lax
from jax.experimental import pallas as pl
from jax.experimental.pallas import tpu as pltpu
from jax.experimental.pallas import tpu_sc as plsc

TOKENS = 8192
DIM = 256
CODES = 8192
TBLK = 4096
NBLK = TOKENS // TBLK
LANES = 128
CBLK = 256
NCHUNK = CODES // CBLK

SC_CORES = 2
SC_SUBCORES = 16
SC_WORKERS = SC_CORES * SC_SUBCORES
ROWS_PER_WORKER = TOKENS // SC_WORKERS
IDX_CHUNKS = ROWS_PER_WORKER // LANES


def _argmin_body(flat_ref, emb_ref, embcol_ref, idx_ref, embt_ref,
                 esq_ref, ebf_ref):
    i = pl.program_id(0)

    @pl.when(i == 0)
    def _init():
        e = emb_ref[...]
        esq_ref[...] = jnp.sum(e * e, axis=0, keepdims=True)
        ebf_ref[...] = e.astype(jnp.bfloat16)

    eblk = embcol_ref[...]
    embt_ref[...] = (
        jnp.transpose(eblk).astype(jnp.bfloat16).astype(jnp.float32))

    f = flat_ref[...]
    sq = f * f
    fsq = jnp.sum(sq[:, :LANES] + sq[:, LANES:], axis=1, keepdims=True)

    fbf2 = f.astype(jnp.bfloat16) * jnp.bfloat16(2.0)
    esq = esq_ref[...]

    runval = None
    runchunk = None
    for c in range(NCHUNK):
        sl = slice(c * CBLK, (c + 1) * CBLK)
        sim2 = jnp.dot(fbf2, ebf_ref[:, sl],
                       preferred_element_type=jnp.float32)
        d = (fsq + esq[:, sl]) - sim2
        if c == 0:
            runval = d
            runchunk = jnp.zeros((TBLK, CBLK), jnp.int32)
        else:
            m = d < runval
            runval = jnp.minimum(runval, d)
            runchunk = jnp.where(m, jnp.int32(c), runchunk)

    lane = lax.broadcasted_iota(jnp.int32, (TBLK, CBLK), 1)
    runidx = runchunk * CBLK + lane
    fmin = jnp.min(runval, axis=1, keepdims=True)
    cand = jnp.where(runval == fmin, runidx, jnp.int32(0x7FFFFFFF))
    fidx = jnp.min(cand, axis=1, keepdims=True)
    idx_ref[...] = fidx.reshape(TBLK // ROWS_PER_WORKER, IDX_CHUNKS, LANES)


@jax.jit
def _argmin_call(flat, embeddings):
    return pl.pallas_call(
        _argmin_body,
        grid=(NBLK,),
        in_specs=[
            pl.BlockSpec((TBLK, DIM), lambda i: (i, 0)),
            pl.BlockSpec((DIM, CODES), lambda i: (0, 0)),
            pl.BlockSpec((DIM, TBLK), lambda i: (0, i)),
        ],
        out_specs=[
            pl.BlockSpec((TBLK // ROWS_PER_WORKER, IDX_CHUNKS, LANES),
                         lambda i: (i, 0, 0)),
            pl.BlockSpec((TBLK, DIM), lambda i: (i, 0)),
        ],
        out_shape=[
            jax.ShapeDtypeStruct((SC_WORKERS, IDX_CHUNKS, LANES), jnp.int32),
            jax.ShapeDtypeStruct((TOKENS, DIM), jnp.float32),
        ],
        scratch_shapes=[
            pltpu.VMEM((1, CODES), jnp.float32),
            pltpu.VMEM((DIM, CODES), jnp.bfloat16),
        ],
    )(flat, embeddings, embeddings)


def _gather_call(embt, idx):
    mesh = plsc.VectorSubcoreMesh(core_axis_name="c", subcore_axis_name="s")

    @functools.partial(
        pl.kernel,
        mesh=mesh,
        out_type=jax.ShapeDtypeStruct((TOKENS, DIM), jnp.float32),
        scratch_types=[
            pltpu.VMEM((IDX_CHUNKS, LANES), jnp.int32),
            pltpu.VMEM((ROWS_PER_WORKER, DIM), jnp.float32),
            pltpu.SemaphoreType.DMA,
        ],
    )
    def k(table_hbm, idx_hbm, out_hbm, idx_v, rows_v, sem):
        wid = lax.axis_index("s") * SC_CORES + lax.axis_index("c")
        pltpu.sync_copy(idx_hbm.at[wid], idx_v)
        copies = [
            pltpu.async_copy(table_hbm.at[idx_v.at[j]],
                             rows_v.at[pl.ds(j * LANES, LANES)], sem)
            for j in range(IDX_CHUNKS)
        ]
        for cp in copies:
            cp.wait()
        pltpu.sync_copy(rows_v, out_hbm.at[pl.ds(wid * ROWS_PER_WORKER,
                                                 ROWS_PER_WORKER)])

    return k(embt, idx)


def kernel(x, embeddings):
    flat = x.reshape(TOKENS, DIM)
    idx, embt = _argmin_call(flat, embeddings)
    quantized = _gather_call(embt, idx)
    return quantized.reshape(x.shape)
